# Initial kernel scaffold; baseline (speedup 1.0000x reference)
import jax
import jax.numpy as jnp
from jax.experimental import pallas as pl


def kernel(af, edge_index, bf, W_edge, b_edge, W_attn, b_attn, W_node, b_node):
    raise NotImplementedError("write your pallas kernel here")


# speedup vs baseline: 3.5883x; 3.5883x over previous
import functools

import jax
import jax.numpy as jnp
from jax import lax
from jax.experimental import pallas as pl
from jax.experimental.pallas import tpu as pltpu
from jax.experimental.pallas import tpu_sc as plsc

F32 = jnp.float32

N = 10000
E = 320000
DF = 128
DE = 16

NC = 2
NS = 16
L = 16
NW = NC * NS
E_PAD = 327680
EPW = E_PAD // NW
CHUNK = 640
NCH = EPW // CHUNK
G = 80
NG = CHUNK // G
AGG_R = 10240
RPT = AGG_R // NS


def _tables_body(af_ref, w_ref, t_ref):
    t_ref[...] = jnp.dot(af_ref[...], w_ref[...], preferred_element_type=F32)


def _node_tables(af, wcat):
    return pl.pallas_call(
        _tables_body,
        grid=(5,),
        in_specs=[pl.BlockSpec((N // 5, DF), lambda i: (i, 0)),
                  pl.BlockSpec((DF, 34), lambda i: (0, 0))],
        out_specs=pl.BlockSpec((N // 5, 34), lambda i: (i, 0)),
        out_shape=jax.ShapeDtypeStruct((N, 34), F32),
    )(af, wcat)


def _edgebase_body(bf_ref, w_ref, b_ref, o16_ref, oa_ref):
    y = jnp.dot(bf_ref[...], w_ref[...], preferred_element_type=F32) + b_ref[...]
    o16_ref[...] = y[:, :DE]
    oa_ref[...] = y[:, DE:DE + 1]


def _edge_base(bf, w17, b17):
    R = 4000
    return pl.pallas_call(
        _edgebase_body,
        grid=(E // R,),
        in_specs=[pl.BlockSpec((R, DE), lambda i: (i, 0)),
                  pl.BlockSpec((DE, DE + 1), lambda i: (0, 0)),
                  pl.BlockSpec((1, DE + 1), lambda i: (0, 0))],
        out_specs=[pl.BlockSpec((R, DE), lambda i: (i, 0)),
                   pl.BlockSpec((R, 1), lambda i: (i, 0))],
        out_shape=[jax.ShapeDtypeStruct((E, DE), F32),
                   jax.ShapeDtypeStruct((E, 1), F32)],
    )(bf, w17, b17)


def _node_body(af_ref, agg_ref, w1_ref, w2_ref, b_ref, o_ref):
    s = agg_ref[0] + agg_ref[1]
    y = (jnp.dot(af_ref[...], w1_ref[...], preferred_element_type=F32)
         + jnp.dot(s, w2_ref[...], preferred_element_type=F32) + b_ref[...])
    o_ref[...] = jnp.maximum(y, 0.0)


def _node_mlp(af, aggp, w1, w2, b):
    R = N // 5
    return pl.pallas_call(
        _node_body,
        grid=(5,),
        in_specs=[pl.BlockSpec((R, DF), lambda i: (i, 0)),
                  pl.BlockSpec((NC, R, DE), lambda i: (0, i, 0)),
                  pl.BlockSpec((DF, DF), lambda i: (0, 0)),
                  pl.BlockSpec((DE, DF), lambda i: (0, 0)),
                  pl.BlockSpec((1, DF), lambda i: (0, 0))],
        out_specs=pl.BlockSpec((R, DF), lambda i: (i, 0)),
        out_shape=jax.ShapeDtypeStruct((N, DF), F32),
    )(af, aggp, w1, w2, b)


def _sc_edge_body(src2, dst2, p16, q16, sa, da, b16, ba,
                  ubf_out, agg_out,
                  sa_v, da_v, si_v, di_v, p_v, q_v, b_v, ba_v, at_v,
                  ubf_v, msg_v, z_v, agg_sh, sem_g):
    cid = lax.axis_index("c")
    sid = lax.axis_index("s")
    wid = sid * NC + cid

    pltpu.sync_copy(sa, sa_v)
    pltpu.sync_copy(da, da_v)

    def _zrow(i, _):
        z_v[i, :] = jnp.zeros((L,), F32)
        return 0
    lax.fori_loop(0, RPT, _zrow, 0)
    pltpu.sync_copy(z_v, agg_sh.at[pl.ds(sid * RPT, RPT)])
    plsc.subcore_barrier()

    def _chunk(ci, _):
        base = wid * EPW + ci * CHUNK
        rb = wid * (EPW // G) + ci * NG
        pltpu.sync_copy(src2.at[pl.ds(rb, NG)], si_v)
        pltpu.sync_copy(dst2.at[pl.ds(rb, NG)], di_v)
        gds = []
        for g in range(NG):
            gds.append(pltpu.async_copy(
                p16.at[si_v.at[g]], p_v.at[pl.ds(g * G, G)], sem_g))
            gds.append(pltpu.async_copy(
                q16.at[di_v.at[g]], q_v.at[pl.ds(g * G, G)], sem_g))
        pltpu.sync_copy(b16.at[pl.ds(base, CHUNK)], b_v)
        pltpu.sync_copy(ba.at[pl.ds(base, CHUNK)], ba_v)
        for g in range(NG):
            for j in range(G // L):
                off = g * G + j * L
                sv = si_v[g, pl.ds(j * L, L)]
                dv = di_v[g, pl.ds(j * L, L)]
                pa = (plsc.load_gather(sa_v, [sv])
                      + plsc.load_gather(da_v, [dv])
                      + ba_v[pl.ds(off, L)])
                at_v[pl.ds(off, L)] = 1.0 / (1.0 + jnp.exp(-pa))
        for d in gds:
            d.wait()

        def _row(i, _):
            pre = p_v[i, :] + q_v[i, :] + b_v[i, :]
            u = jnp.maximum(pre, 0.0)
            ubf_v[i, :] = u
            av = plsc.load_gather(at_v, [jnp.broadcast_to(i, (L,))])
            msg_v[i, :] = u * av
            return 0
        lax.fori_loop(0, CHUNK, _row, 0)

        pltpu.sync_copy(ubf_v, ubf_out.at[pl.ds(base, CHUNK)])
        for g in range(NG):
            pltpu.sync_copy(msg_v.at[pl.ds(g * G, G)],
                            agg_sh.at[di_v.at[g]], add=True)
        return 0
    n_real = jnp.minimum(NCH, (E - wid * EPW) // CHUNK)
    lax.fori_loop(0, n_real, _chunk, 0)

    plsc.subcore_barrier()
    pltpu.sync_copy(agg_sh.at[pl.ds(sid * RPT, RPT)], z_v)
    pltpu.sync_copy(z_v, agg_out.at[pl.ds(cid * AGG_R + sid * RPT, RPT)])


@functools.partial(jax.jit, static_argnums=())
def _sc_edge(src2, dst2, p16, q16, sa, da, b16, ba):
    mesh = plsc.VectorSubcoreMesh(core_axis_name="c", subcore_axis_name="s")
    return pl.kernel(
        _sc_edge_body,
        out_type=(jax.ShapeDtypeStruct((E, DE), F32),
                  jax.ShapeDtypeStruct((NC * AGG_R, DE), F32)),
        mesh=mesh,
        compiler_params=pltpu.CompilerParams(use_tc_tiling_on_sc=False,
                                             needs_layout_passes=False),
        scratch_types=[
            pltpu.VMEM((N,), F32),
            pltpu.VMEM((N,), F32),
            pltpu.VMEM((NG, G), jnp.int32),
            pltpu.VMEM((NG, G), jnp.int32),
            pltpu.VMEM((CHUNK, DE), F32),
            pltpu.VMEM((CHUNK, DE), F32),
            pltpu.VMEM((CHUNK, DE), F32),
            pltpu.VMEM((CHUNK,), F32),
            pltpu.VMEM((CHUNK,), F32),
            pltpu.VMEM((CHUNK, DE), F32),
            pltpu.VMEM((CHUNK, DE), F32),
            pltpu.VMEM((RPT, DE), F32),
            pltpu.VMEM_SHARED((AGG_R, DE), F32),
            pltpu.SemaphoreType.DMA,
        ],
    )(src2, dst2, p16, q16, sa, da, b16, ba)


def kernel(af, edge_index, bf, W_edge, b_edge, W_attn, b_attn, W_node, b_node):
    src = edge_index[0]
    dst = edge_index[1]
    wcat = jnp.concatenate(
        [W_edge[:DF], W_attn[:DF], W_edge[DF:2 * DF], W_attn[DF:2 * DF]],
        axis=1)
    t = _node_tables(af, wcat)
    p16 = t[:, 0:DE]
    sa = t[:, DE]
    q16 = t[:, DE + 1:2 * DE + 1]
    da = t[:, 2 * DE + 1]

    w17 = jnp.concatenate([W_edge[2 * DF:], W_attn[2 * DF:]], axis=1)
    b17 = jnp.concatenate([b_edge, b_attn])[None, :]
    b16e, bae = _edge_base(bf, w17, b17)

    pad = jnp.zeros((E_PAD - E,), jnp.int32)
    src2 = jnp.concatenate([src, pad]).reshape(E_PAD // G, G)
    dst2 = jnp.concatenate([dst, pad]).reshape(E_PAD // G, G)
    ubf, aggf = _sc_edge(src2, dst2, p16, q16, sa, da, b16e,
                         bae.reshape(E))
    aggp = aggf.reshape(NC, AGG_R, DE)[:, :N]

    uaf = _node_mlp(af, aggp, W_node[:DF], W_node[DF:], b_node[None, :])
    return (uaf, ubf)


# speedup vs baseline: 5.6450x; 1.5732x over previous
import functools

import jax
import jax.numpy as jnp
from jax import lax
from jax.experimental import pallas as pl
from jax.experimental.pallas import tpu as pltpu
from jax.experimental.pallas import tpu_sc as plsc

F32 = jnp.float32

N = 10000
E = 320000
DF = 128
DE = 16

NC = 2
NS = 16
L = 16
NW = NC * NS
CHUNK = 640
G = 80
NG = CHUNK // G
NBLK = E // CHUNK
AGG_R = 10240
RPT = AGG_R // NS


def _tables_body(af_ref, w_ref, p_ref, q_ref, sd_ref):
    y = jnp.dot(af_ref[...], w_ref[...], preferred_element_type=F32)
    p_ref[...] = y[:, :DE]
    q_ref[...] = y[:, DE:2 * DE]
    sd_ref[...] = y[:, 2 * DE:]


def _node_tables(af, wcat):
    return pl.pallas_call(
        _tables_body,
        grid=(5,),
        in_specs=[pl.BlockSpec((N // 5, DF), lambda i: (i, 0)),
                  pl.BlockSpec((DF, 34), lambda i: (0, 0))],
        out_specs=[pl.BlockSpec((N // 5, DE), lambda i: (i, 0)),
                   pl.BlockSpec((N // 5, DE), lambda i: (i, 0)),
                   pl.BlockSpec((N // 5, 2), lambda i: (i, 0))],
        out_shape=[jax.ShapeDtypeStruct((N, DE), F32),
                   jax.ShapeDtypeStruct((N, DE), F32),
                   jax.ShapeDtypeStruct((N, 2), F32)],
    )(af, wcat)


def _edgebase_body(bf_ref, wb_ref, wa_ref, bb_ref, ba_ref, ob_ref, oa_ref):
    x = bf_ref[...]
    ob_ref[...] = (jnp.dot(x, wb_ref[...], preferred_element_type=F32)
                   + bb_ref[...])
    oa_ref[...] = (jnp.dot(x, wa_ref[...], preferred_element_type=F32)
                   + ba_ref[...])


def _edge_base(bf8, w8b, w8a, bb, ba):
    R = 800
    E8 = E // 8
    return pl.pallas_call(
        _edgebase_body,
        grid=(E8 // R,),
        in_specs=[pl.BlockSpec((R, 128), lambda i: (i, 0)),
                  pl.BlockSpec((128, 128), lambda i: (0, 0)),
                  pl.BlockSpec((128, 8), lambda i: (0, 0)),
                  pl.BlockSpec((1, 128), lambda i: (0, 0)),
                  pl.BlockSpec((1, 8), lambda i: (0, 0))],
        out_specs=[pl.BlockSpec((R, 128), lambda i: (i, 0)),
                   pl.BlockSpec((R, 8), lambda i: (i, 0))],
        out_shape=[jax.ShapeDtypeStruct((E8, 128), F32),
                   jax.ShapeDtypeStruct((E8, 8), F32)],
    )(bf8, w8b, w8a, bb, ba)


def _node_body(af_ref, agg_ref, w1_ref, w2_ref, b_ref, o_ref):
    s = agg_ref[0] + agg_ref[1]
    y = (jnp.dot(af_ref[...], w1_ref[...], preferred_element_type=F32)
         + jnp.dot(s, w2_ref[...], preferred_element_type=F32) + b_ref[...])
    o_ref[...] = jnp.maximum(y, 0.0)


def _node_mlp(af, aggp, w1, w2, b):
    R = N // 5
    return pl.pallas_call(
        _node_body,
        grid=(5,),
        in_specs=[pl.BlockSpec((R, DF), lambda i: (i, 0)),
                  pl.BlockSpec((NC, R, DE), lambda i: (0, i, 0)),
                  pl.BlockSpec((DF, DF), lambda i: (0, 0)),
                  pl.BlockSpec((DE, DF), lambda i: (0, 0)),
                  pl.BlockSpec((1, DF), lambda i: (0, 0))],
        out_specs=pl.BlockSpec((R, DF), lambda i: (i, 0)),
        out_shape=jax.ShapeDtypeStruct((N, DF), F32),
    )(af, aggp, w1, w2, b)


def _sc_edge_body(src2, dst2, p16, q16, sa, da, b8, bae,
                  ubf_out, agg_out,
                  sa_v, da_v, si_v, di_v, p_v, q_v, b_v, ba_v, at_v,
                  ubf_v, msg_v, z_v, agg_sh, sem_g):
    cid = lax.axis_index("c")
    sid = lax.axis_index("s")
    wid = sid * NC + cid

    pltpu.sync_copy(sa, sa_v)
    pltpu.sync_copy(da, da_v)

    def _zrow(i, _):
        z_v[i, :] = jnp.zeros((L,), F32)
        return 0
    lax.fori_loop(0, RPT, _zrow, 0)
    pltpu.sync_copy(z_v, agg_sh.at[pl.ds(sid * RPT, RPT)])
    plsc.subcore_barrier()

    iota = lax.iota(jnp.int32, L)

    def _chunk(ci, _):
        blk = wid + NW * ci
        base = blk * CHUNK
        pltpu.sync_copy(src2.at[pl.ds(blk * NG, NG)], si_v)
        pltpu.sync_copy(dst2.at[pl.ds(blk * NG, NG)], di_v)
        gds = []
        for g in range(NG):
            gds.append(pltpu.async_copy(
                p16.at[si_v.at[g]], p_v.at[pl.ds(g * G, G)], sem_g))
            gds.append(pltpu.async_copy(
                q16.at[di_v.at[g]], q_v.at[pl.ds(g * G, G)], sem_g))
        pltpu.sync_copy(b8.at[pl.ds(blk * (CHUNK // 8), CHUNK // 8)], b_v)
        pltpu.sync_copy(bae.at[pl.ds(base, CHUNK)], ba_v)
        for g in range(NG):
            for j in range(G // L):
                off = g * G + j * L
                sv = si_v[g, pl.ds(j * L, L)]
                dv = di_v[g, pl.ds(j * L, L)]
                pa = (plsc.load_gather(sa_v, [sv])
                      + plsc.load_gather(da_v, [dv])
                      + ba_v[pl.ds(off, L)])
                at_v[pl.ds(off, L)] = 1.0 / (1.0 + jnp.exp(-pa))
        for d in gds:
            d.wait()

        def _row(r, _):
            for k in range(8):
                i = r * 8 + k
                pre = (p_v[i, :] + q_v[i, :]
                       + b_v[r, pl.ds(k * DE, DE)])
                u = jnp.maximum(pre, 0.0)
                plsc.store_scatter(
                    ubf_v, [iota // 8, jnp.broadcast_to(i // 128, (L,)),
                            iota % 8, jnp.broadcast_to(i % 128, (L,))], u)
                av = plsc.load_gather(at_v, [jnp.broadcast_to(i, (L,))])
                msg_v[i, :] = u * av
            return 0
        lax.fori_loop(0, CHUNK // 8, _row, 0)

        pltpu.sync_copy(ubf_v,
                        ubf_out.at[:, pl.ds(blk * (CHUNK // 128), CHUNK // 128)])
        for g in range(NG):
            pltpu.sync_copy(msg_v.at[pl.ds(g * G, G)],
                            agg_sh.at[di_v.at[g]], add=True)
        return 0
    n_w = (NBLK - 1 - wid) // NW + 1
    lax.fori_loop(0, n_w, _chunk, 0)

    plsc.subcore_barrier()
    pltpu.sync_copy(agg_sh.at[pl.ds(sid * RPT, RPT)], z_v)
    pltpu.sync_copy(z_v, agg_out.at[pl.ds(cid * AGG_R + sid * RPT, RPT)])


@functools.partial(jax.jit, static_argnums=())
def _sc_edge(src2, dst2, p16, q16, sa, da, b8, bae):
    mesh = plsc.VectorSubcoreMesh(core_axis_name="c", subcore_axis_name="s")
    return pl.kernel(
        _sc_edge_body,
        out_type=(jax.ShapeDtypeStruct((2, E // 128, 8, 128), F32),
                  jax.ShapeDtypeStruct((NC * AGG_R, DE), F32)),
        mesh=mesh,
        compiler_params=pltpu.CompilerParams(use_tc_tiling_on_sc=False,
                                             needs_layout_passes=False),
        scratch_types=[
            pltpu.VMEM((N,), F32),
            pltpu.VMEM((N,), F32),
            pltpu.VMEM((NG, G), jnp.int32),
            pltpu.VMEM((NG, G), jnp.int32),
            pltpu.VMEM((CHUNK, DE), F32),
            pltpu.VMEM((CHUNK, DE), F32),
            pltpu.VMEM((CHUNK // 8, 128), F32),
            pltpu.VMEM((CHUNK,), F32),
            pltpu.VMEM((CHUNK,), F32),
            pltpu.VMEM((2, CHUNK // 128, 8, 128), F32),
            pltpu.VMEM((CHUNK, DE), F32),
            pltpu.VMEM((RPT, DE), F32),
            pltpu.VMEM_SHARED((AGG_R, DE), F32),
            pltpu.SemaphoreType.DMA,
        ],
    )(src2, dst2, p16, q16, sa, da, b8, bae)


def kernel(af, edge_index, bf, W_edge, b_edge, W_attn, b_attn, W_node, b_node):
    wcat = jnp.concatenate(
        [W_edge[:DF], W_edge[DF:2 * DF], W_attn[:DF], W_attn[DF:2 * DF]],
        axis=1)
    p16, q16, sda = _node_tables(af, wcat)

    w16 = W_edge[2 * DF:]
    wa = W_attn[2 * DF:]
    zero = jnp.zeros_like(w16)
    w8b = jnp.concatenate(
        [jnp.concatenate([w16 if i == j else zero for j in range(8)], axis=1)
         for i in range(8)], axis=0)
    zcol = jnp.zeros_like(wa)
    w8a = jnp.concatenate(
        [jnp.concatenate([wa if i == j else zcol for j in range(8)], axis=1)
         for i in range(8)], axis=0)
    bb = jnp.tile(b_edge, 8)[None, :]
    ba = jnp.tile(b_attn, 8)[None, :]
    bf8 = bf.reshape(E // 8, 128)
    b8, a8 = _edge_base(bf8, w8b, w8a, bb, ba)

    src2 = edge_index[0].reshape(E // G, G)
    dst2 = edge_index[1].reshape(E // G, G)
    sa = sda[:, 0]
    da = sda[:, 1]
    bae = a8.reshape(E // 8 * 8)
    ubf4, aggf = _sc_edge(src2, dst2, p16, q16, sa, da, b8, bae)
    ubf = ubf4.transpose(1, 3, 0, 2).reshape(E, DE)
    aggp = aggf.reshape(NC, AGG_R, DE)[:, :N]

    uaf = _node_mlp(af, aggp, W_node[:DF], W_node[DF:], b_node[None, :])
    return (uaf, ubf)


# speedup vs baseline: 5.7957x; 1.0267x over previous
import functools

import jax
import jax.numpy as jnp
from jax import lax
from jax.experimental import pallas as pl
from jax.experimental.pallas import tpu as pltpu
from jax.experimental.pallas import tpu_sc as plsc

F32 = jnp.float32

N = 10000
E = 320000
DF = 128
DE = 16

NC = 2
NS = 16
L = 16
NW = NC * NS
CHUNK = 640
G = 80
NG = CHUNK // G
NBLK = E // CHUNK
AGG_R = 10240
RPT = AGG_R // NS


def _tables_body(af_ref, w_ref, p_ref, q_ref, sd_ref):
    y = jnp.dot(af_ref[...], w_ref[...], preferred_element_type=F32)
    p_ref[...] = y[:, :DE]
    q_ref[...] = y[:, DE:2 * DE]
    sd_ref[...] = y[:, 2 * DE:]


def _node_tables(af, wcat):
    return pl.pallas_call(
        _tables_body,
        grid=(5,),
        in_specs=[pl.BlockSpec((N // 5, DF), lambda i: (i, 0)),
                  pl.BlockSpec((DF, 34), lambda i: (0, 0))],
        out_specs=[pl.BlockSpec((N // 5, DE), lambda i: (i, 0)),
                   pl.BlockSpec((N // 5, DE), lambda i: (i, 0)),
                   pl.BlockSpec((N // 5, 2), lambda i: (i, 0))],
        out_shape=[jax.ShapeDtypeStruct((N, DE), F32),
                   jax.ShapeDtypeStruct((N, DE), F32),
                   jax.ShapeDtypeStruct((N, 2), F32)],
    )(af, wcat)


def _edgebase_body(bf_ref, wb_ref, wa_ref, bb_ref, ba_ref, ob_ref, oa_ref):
    x = bf_ref[...]
    ob_ref[...] = (jnp.dot(x, wb_ref[...], preferred_element_type=F32)
                   + bb_ref[...])
    oa_ref[...] = (jnp.dot(x, wa_ref[...], preferred_element_type=F32)
                   + ba_ref[...])


def _edge_base(bf8, w8b, w8a, bb, ba):
    R = 800
    E8 = E // 8
    return pl.pallas_call(
        _edgebase_body,
        grid=(E8 // R,),
        in_specs=[pl.BlockSpec((R, 128), lambda i: (i, 0)),
                  pl.BlockSpec((128, 128), lambda i: (0, 0)),
                  pl.BlockSpec((128, 8), lambda i: (0, 0)),
                  pl.BlockSpec((1, 128), lambda i: (0, 0)),
                  pl.BlockSpec((1, 8), lambda i: (0, 0))],
        out_specs=[pl.BlockSpec((R, 128), lambda i: (i, 0)),
                   pl.BlockSpec((R, 8), lambda i: (i, 0))],
        out_shape=[jax.ShapeDtypeStruct((E8, 128), F32),
                   jax.ShapeDtypeStruct((E8, 8), F32)],
    )(bf8, w8b, w8a, bb, ba)


def _node_body(af_ref, agg_ref, w1_ref, w2_ref, b_ref, o_ref):
    s = agg_ref[0] + agg_ref[1]
    y = (jnp.dot(af_ref[...], w1_ref[...], preferred_element_type=F32)
         + jnp.dot(s, w2_ref[...], preferred_element_type=F32) + b_ref[...])
    o_ref[...] = jnp.maximum(y, 0.0)


def _node_mlp(af, aggp, w1, w2, b):
    R = N // 5
    return pl.pallas_call(
        _node_body,
        grid=(5,),
        in_specs=[pl.BlockSpec((R, DF), lambda i: (i, 0)),
                  pl.BlockSpec((NC, R, DE), lambda i: (0, i, 0)),
                  pl.BlockSpec((DF, DF), lambda i: (0, 0)),
                  pl.BlockSpec((DE, DF), lambda i: (0, 0)),
                  pl.BlockSpec((1, DF), lambda i: (0, 0))],
        out_specs=pl.BlockSpec((R, DF), lambda i: (i, 0)),
        out_shape=jax.ShapeDtypeStruct((N, DF), F32),
    )(af, aggp, w1, w2, b)


def _sc_edge_body(src2, dst2, p16, q16, sa, da, b8, bae,
                  ubf_out, agg_out,
                  sa_v, da_v, si_v, di_v, p_v, q_v, b_v, ba_v, at_v,
                  ubf_v, msg_v, z_v, agg_sh, sem_g, sem_b, sem_w):
    cid = lax.axis_index("c")
    sid = lax.axis_index("s")
    wid = sid * NC + cid

    pltpu.sync_copy(sa, sa_v)
    pltpu.sync_copy(da, da_v)

    def _zrow(i, _):
        z_v[i, :] = jnp.zeros((L,), F32)
        return 0
    lax.fori_loop(0, RPT, _zrow, 0)
    pltpu.sync_copy(z_v, agg_sh.at[pl.ds(sid * RPT, RPT)])
    plsc.subcore_barrier()

    iota = lax.iota(jnp.int32, L)

    def _chunk(ci, _):
        blk = wid + NW * ci
        base = blk * CHUNK
        pltpu.sync_copy(src2.at[pl.ds(blk * NG, NG)], si_v)
        pltpu.sync_copy(dst2.at[pl.ds(blk * NG, NG)], di_v)
        gds = []
        for g in range(NG):
            gds.append(pltpu.async_copy(
                p16.at[si_v.at[g]], p_v.at[pl.ds(g * G, G)], sem_g))
            gds.append(pltpu.async_copy(
                q16.at[di_v.at[g]], q_v.at[pl.ds(g * G, G)], sem_g))
        d_b = pltpu.async_copy(
            b8.at[pl.ds(blk * (CHUNK // 8), CHUNK // 8)], b_v, sem_b)
        d_ba = pltpu.async_copy(bae.at[pl.ds(base, CHUNK)], ba_v, sem_b)
        d_b.wait()
        d_ba.wait()
        for g in range(NG):
            for j in range(G // L):
                off = g * G + j * L
                sv = si_v[g, pl.ds(j * L, L)]
                dv = di_v[g, pl.ds(j * L, L)]
                pa = (plsc.load_gather(sa_v, [sv])
                      + plsc.load_gather(da_v, [dv])
                      + ba_v[pl.ds(off, L)])
                at_v[pl.ds(off, L)] = 1.0 / (1.0 + jnp.exp(-pa))
        for d in gds:
            d.wait()

        def _row(r, _):
            for k in range(8):
                i = r * 8 + k
                pre = (p_v[i, :] + q_v[i, :]
                       + b_v[r, pl.ds(k * DE, DE)])
                u = jnp.maximum(pre, 0.0)
                plsc.store_scatter(
                    ubf_v, [iota // 8, jnp.broadcast_to(i // 128, (L,)),
                            iota % 8, jnp.broadcast_to(i % 128, (L,))], u)
                av = plsc.load_gather(at_v, [jnp.broadcast_to(i, (L,))])
                msg_v[i, :] = u * av
            return 0
        lax.fori_loop(0, CHUNK // 8, _row, 0)

        d_u = pltpu.async_copy(
            ubf_v, ubf_out.at[:, pl.ds(blk * (CHUNK // 128), CHUNK // 128)],
            sem_w)
        for g in range(NG):
            pltpu.sync_copy(msg_v.at[pl.ds(g * G, G)],
                            agg_sh.at[di_v.at[g]], add=True)
        d_u.wait()
        return 0
    n_w = (NBLK - 1 - wid) // NW + 1
    lax.fori_loop(0, n_w, _chunk, 0)

    plsc.subcore_barrier()
    pltpu.sync_copy(agg_sh.at[pl.ds(sid * RPT, RPT)], z_v)
    pltpu.sync_copy(z_v, agg_out.at[pl.ds(cid * AGG_R + sid * RPT, RPT)])


@functools.partial(jax.jit, static_argnums=())
def _sc_edge(src2, dst2, p16, q16, sa, da, b8, bae):
    mesh = plsc.VectorSubcoreMesh(core_axis_name="c", subcore_axis_name="s")
    return pl.kernel(
        _sc_edge_body,
        out_type=(jax.ShapeDtypeStruct((2, E // 128, 8, 128), F32),
                  jax.ShapeDtypeStruct((NC * AGG_R, DE), F32)),
        mesh=mesh,
        compiler_params=pltpu.CompilerParams(use_tc_tiling_on_sc=False,
                                             needs_layout_passes=False),
        scratch_types=[
            pltpu.VMEM((N,), F32),
            pltpu.VMEM((N,), F32),
            pltpu.VMEM((NG, G), jnp.int32),
            pltpu.VMEM((NG, G), jnp.int32),
            pltpu.VMEM((CHUNK, DE), F32),
            pltpu.VMEM((CHUNK, DE), F32),
            pltpu.VMEM((CHUNK // 8, 128), F32),
            pltpu.VMEM((CHUNK,), F32),
            pltpu.VMEM((CHUNK,), F32),
            pltpu.VMEM((2, CHUNK // 128, 8, 128), F32),
            pltpu.VMEM((CHUNK, DE), F32),
            pltpu.VMEM((RPT, DE), F32),
            pltpu.VMEM_SHARED((AGG_R, DE), F32),
            pltpu.SemaphoreType.DMA,
            pltpu.SemaphoreType.DMA,
            pltpu.SemaphoreType.DMA,
        ],
    )(src2, dst2, p16, q16, sa, da, b8, bae)


def kernel(af, edge_index, bf, W_edge, b_edge, W_attn, b_attn, W_node, b_node):
    wcat = jnp.concatenate(
        [W_edge[:DF], W_edge[DF:2 * DF], W_attn[:DF], W_attn[DF:2 * DF]],
        axis=1)
    p16, q16, sda = _node_tables(af, wcat)

    w16 = W_edge[2 * DF:]
    wa = W_attn[2 * DF:]
    zero = jnp.zeros_like(w16)
    w8b = jnp.concatenate(
        [jnp.concatenate([w16 if i == j else zero for j in range(8)], axis=1)
         for i in range(8)], axis=0)
    zcol = jnp.zeros_like(wa)
    w8a = jnp.concatenate(
        [jnp.concatenate([wa if i == j else zcol for j in range(8)], axis=1)
         for i in range(8)], axis=0)
    bb = jnp.tile(b_edge, 8)[None, :]
    ba = jnp.tile(b_attn, 8)[None, :]
    bf8 = bf.reshape(E // 8, 128)
    b8, a8 = _edge_base(bf8, w8b, w8a, bb, ba)

    src2 = edge_index[0].reshape(E // G, G)
    dst2 = edge_index[1].reshape(E // G, G)
    sa = sda[:, 0]
    da = sda[:, 1]
    bae = a8.reshape(E // 8 * 8)
    ubf4, aggf = _sc_edge(src2, dst2, p16, q16, sa, da, b8, bae)
    ubf = ubf4.transpose(1, 3, 0, 2).reshape(E, DE)
    aggp = aggf.reshape(NC, AGG_R, DE)[:, :N]

    uaf = _node_mlp(af, aggp, W_node[:DF], W_node[DF:], b_node[None, :])
    return (uaf, ubf)


# speedup vs baseline: 5.8254x; 1.0051x over previous
import functools

import jax
import jax.numpy as jnp
from jax import lax
from jax.experimental import pallas as pl
from jax.experimental.pallas import tpu as pltpu
from jax.experimental.pallas import tpu_sc as plsc

F32 = jnp.float32

N = 10000
E = 320000
DF = 128
DE = 16

NC = 2
NS = 16
L = 16
NW = NC * NS
CHUNK = 640
G = 128
NG = CHUNK // G
NBLK = E // CHUNK
AGG_R = 10240
RPT = AGG_R // NS


def _tables_body(af_ref, w_ref, p_ref, q_ref, sd_ref):
    y = jnp.dot(af_ref[...], w_ref[...], preferred_element_type=F32)
    p_ref[...] = y[:, :DE]
    q_ref[...] = y[:, DE:2 * DE]
    sd_ref[...] = y[:, 2 * DE:]


def _node_tables(af, wcat):
    return pl.pallas_call(
        _tables_body,
        grid=(5,),
        in_specs=[pl.BlockSpec((N // 5, DF), lambda i: (i, 0)),
                  pl.BlockSpec((DF, 34), lambda i: (0, 0))],
        out_specs=[pl.BlockSpec((N // 5, DE), lambda i: (i, 0)),
                   pl.BlockSpec((N // 5, DE), lambda i: (i, 0)),
                   pl.BlockSpec((N // 5, 2), lambda i: (i, 0))],
        out_shape=[jax.ShapeDtypeStruct((N, DE), F32),
                   jax.ShapeDtypeStruct((N, DE), F32),
                   jax.ShapeDtypeStruct((N, 2), F32)],
    )(af, wcat)


def _edgebase_body(bf_ref, wb_ref, wa_ref, bb_ref, ba_ref, ob_ref, oa_ref):
    x = bf_ref[...]
    ob_ref[...] = (jnp.dot(x, wb_ref[...], preferred_element_type=F32)
                   + bb_ref[...])
    oa_ref[...] = (jnp.dot(x, wa_ref[...], preferred_element_type=F32)
                   + ba_ref[...])


def _edge_base(bf8, w8b, w8a, bb, ba):
    R = 800
    E8 = E // 8
    return pl.pallas_call(
        _edgebase_body,
        grid=(E8 // R,),
        in_specs=[pl.BlockSpec((R, 128), lambda i: (i, 0)),
                  pl.BlockSpec((128, 128), lambda i: (0, 0)),
                  pl.BlockSpec((128, 8), lambda i: (0, 0)),
                  pl.BlockSpec((1, 128), lambda i: (0, 0)),
                  pl.BlockSpec((1, 8), lambda i: (0, 0))],
        out_specs=[pl.BlockSpec((R, 128), lambda i: (i, 0)),
                   pl.BlockSpec((R, 8), lambda i: (i, 0))],
        out_shape=[jax.ShapeDtypeStruct((E8, 128), F32),
                   jax.ShapeDtypeStruct((E8, 8), F32)],
    )(bf8, w8b, w8a, bb, ba)


def _node_body(af_ref, agg_ref, w1_ref, w2_ref, b_ref, o_ref):
    s = agg_ref[0] + agg_ref[1]
    y = (jnp.dot(af_ref[...], w1_ref[...], preferred_element_type=F32)
         + jnp.dot(s, w2_ref[...], preferred_element_type=F32) + b_ref[...])
    o_ref[...] = jnp.maximum(y, 0.0)


def _node_mlp(af, aggp, w1, w2, b):
    R = N // 5
    return pl.pallas_call(
        _node_body,
        grid=(5,),
        in_specs=[pl.BlockSpec((R, DF), lambda i: (i, 0)),
                  pl.BlockSpec((NC, R, DE), lambda i: (0, i, 0)),
                  pl.BlockSpec((DF, DF), lambda i: (0, 0)),
                  pl.BlockSpec((DE, DF), lambda i: (0, 0)),
                  pl.BlockSpec((1, DF), lambda i: (0, 0))],
        out_specs=pl.BlockSpec((R, DF), lambda i: (i, 0)),
        out_shape=jax.ShapeDtypeStruct((N, DF), F32),
    )(af, aggp, w1, w2, b)


def _sc_edge_body(src128, dst128, p16, q16, sa, da, b8, bae,
                  ubf_out, agg_out,
                  sa_v, da_v, si_v, di_v, p_v, q_v, b_v, ba_v, at_v,
                  ubf_v, msg_v, z_v, agg_sh, sem_g, sem_b, sem_w):
    cid = lax.axis_index("c")
    sid = lax.axis_index("s")
    wid = sid * NC + cid

    pltpu.sync_copy(sa, sa_v)
    pltpu.sync_copy(da, da_v)

    def _zrow(i, _):
        z_v[i, :] = jnp.zeros((L,), F32)
        return 0
    lax.fori_loop(0, RPT, _zrow, 0)
    pltpu.sync_copy(z_v, agg_sh.at[pl.ds(sid * RPT, RPT)])
    plsc.subcore_barrier()

    iota = lax.iota(jnp.int32, L)

    def _chunk(ci, _):
        blk = wid + NW * ci
        base = blk * CHUNK
        pltpu.sync_copy(src128.at[pl.ds(blk * NG, NG)], si_v)
        pltpu.sync_copy(dst128.at[pl.ds(blk * NG, NG)], di_v)
        gds = []
        for g in range(NG):
            gds.append(pltpu.async_copy(
                p16.at[si_v.at[g]], p_v.at[pl.ds(g * G, G)], sem_g))
            gds.append(pltpu.async_copy(
                q16.at[di_v.at[g]], q_v.at[pl.ds(g * G, G)], sem_g))
        d_b = pltpu.async_copy(
            b8.at[pl.ds(blk * (CHUNK // 8), CHUNK // 8)], b_v, sem_b)
        d_ba = pltpu.async_copy(bae.at[pl.ds(base, CHUNK)], ba_v, sem_b)
        d_b.wait()
        d_ba.wait()
        for g in range(NG):
            for j in range(G // L):
                off = g * G + j * L
                sv = si_v[g, pl.ds(j * L, L)]
                dv = di_v[g, pl.ds(j * L, L)]
                pa = (plsc.load_gather(sa_v, [sv])
                      + plsc.load_gather(da_v, [dv])
                      + ba_v[pl.ds(off, L)])
                at_v[pl.ds(off, L)] = 1.0 / (1.0 + jnp.exp(-pa))
        for d in gds:
            d.wait()

        def _row(r, _):
            for k in range(8):
                i = r * 8 + k
                pre = (p_v[i, :] + q_v[i, :]
                       + b_v[r, pl.ds(k * DE, DE)])
                u = jnp.maximum(pre, 0.0)
                plsc.store_scatter(
                    ubf_v, [iota // 8, jnp.broadcast_to(i // 128, (L,)),
                            iota % 8, jnp.broadcast_to(i % 128, (L,))], u)
                av = plsc.load_gather(at_v, [jnp.broadcast_to(i, (L,))])
                msg_v[i, :] = u * av
            return 0
        lax.fori_loop(0, CHUNK // 8, _row, 0)

        d_u = pltpu.async_copy(
            ubf_v, ubf_out.at[:, pl.ds(blk * (CHUNK // 128), CHUNK // 128)],
            sem_w)
        for g in range(NG):
            pltpu.sync_copy(msg_v.at[pl.ds(g * G, G)],
                            agg_sh.at[di_v.at[g]], add=True)
        d_u.wait()
        return 0
    n_w = (NBLK - 1 - wid) // NW + 1
    lax.fori_loop(0, n_w, _chunk, 0)

    plsc.subcore_barrier()
    pltpu.sync_copy(agg_sh.at[pl.ds(sid * RPT, RPT)], z_v)
    pltpu.sync_copy(z_v, agg_out.at[pl.ds(cid * AGG_R + sid * RPT, RPT)])


@functools.partial(jax.jit, static_argnums=())
def _sc_edge(src128, dst128, p16, q16, sa, da, b8, bae):
    mesh = plsc.VectorSubcoreMesh(core_axis_name="c", subcore_axis_name="s")
    return pl.kernel(
        _sc_edge_body,
        out_type=(jax.ShapeDtypeStruct((2, E // 128, 8, 128), F32),
                  jax.ShapeDtypeStruct((NC * AGG_R, DE), F32)),
        mesh=mesh,
        compiler_params=pltpu.CompilerParams(use_tc_tiling_on_sc=False,
                                             needs_layout_passes=False),
        scratch_types=[
            pltpu.VMEM((N,), F32),
            pltpu.VMEM((N,), F32),
            pltpu.VMEM((NG, G), jnp.int32),
            pltpu.VMEM((NG, G), jnp.int32),
            pltpu.VMEM((CHUNK, DE), F32),
            pltpu.VMEM((CHUNK, DE), F32),
            pltpu.VMEM((CHUNK // 8, 128), F32),
            pltpu.VMEM((CHUNK,), F32),
            pltpu.VMEM((CHUNK,), F32),
            pltpu.VMEM((2, CHUNK // 128, 8, 128), F32),
            pltpu.VMEM((CHUNK, DE), F32),
            pltpu.VMEM((RPT, DE), F32),
            pltpu.VMEM_SHARED((AGG_R, DE), F32),
            pltpu.SemaphoreType.DMA,
            pltpu.SemaphoreType.DMA,
            pltpu.SemaphoreType.DMA,
        ],
    )(src128, dst128, p16, q16, sa, da, b8, bae)


def kernel(af, edge_index, bf, W_edge, b_edge, W_attn, b_attn, W_node, b_node):
    wcat = jnp.concatenate(
        [W_edge[:DF], W_edge[DF:2 * DF], W_attn[:DF], W_attn[DF:2 * DF]],
        axis=1)
    p16, q16, sda = _node_tables(af, wcat)

    w16 = W_edge[2 * DF:]
    wa = W_attn[2 * DF:]
    zero = jnp.zeros_like(w16)
    w8b = jnp.concatenate(
        [jnp.concatenate([w16 if i == j else zero for j in range(8)], axis=1)
         for i in range(8)], axis=0)
    zcol = jnp.zeros_like(wa)
    w8a = jnp.concatenate(
        [jnp.concatenate([wa if i == j else zcol for j in range(8)], axis=1)
         for i in range(8)], axis=0)
    bb = jnp.tile(b_edge, 8)[None, :]
    ba = jnp.tile(b_attn, 8)[None, :]
    bf8 = bf.reshape(E // 8, 128)
    b8, a8 = _edge_base(bf8, w8b, w8a, bb, ba)

    src2 = edge_index[0].reshape(E // G, G)
    dst2 = edge_index[1].reshape(E // G, G)
    sa = sda[:, 0]
    da = sda[:, 1]
    bae = a8.reshape(E // 8 * 8)
    ubf4, aggf = _sc_edge(src2, dst2, p16, q16, sa, da, b8, bae)
    ubf = ubf4.transpose(1, 3, 0, 2).reshape(E, DE)
    aggp = aggf.reshape(NC, AGG_R, DE)[:, :N]

    uaf = _node_mlp(af, aggp, W_node[:DF], W_node[DF:], b_node[None, :])
    return (uaf, ubf)
